# Initial kernel scaffold; baseline (speedup 1.0000x reference)
import jax
import jax.numpy as jnp
from jax.experimental import pallas as pl


def kernel(x, edge_index, W1, b1, W2, b2, W3, b3, Wd1, bd1, Wd2, bd2, Wd3, bd3, Wd4, bd4):
    raise NotImplementedError("write your pallas kernel here")


# speedup vs baseline: 2.6433x; 2.6433x over previous
import functools

import jax
import jax.numpy as jnp
from jax.experimental import pallas as pl
from jax.experimental.pallas import tpu as pltpu

N = 10000
E = 320000
EMB = 64
HID = 64
ALPHA = 0.1

HEAD_BK = 25600
HEAD_G = (N * EMB) // HEAD_BK


def _head_body(flat_ref, wd1_ref, bd1_ref, wd2_ref, bd2_ref, wd3_ref, bd3_ref,
               wd4_ref, bd4_ref, out_ref, acc_ref):
    g = pl.program_id(0)

    @pl.when(g == 0)
    def _():
        acc_ref[...] = jnp.zeros_like(acc_ref)

    part = jax.lax.dot_general(
        flat_ref[0], wd1_ref[...],
        dimension_numbers=(((1,), (0,)), ((), ())),
        preferred_element_type=jnp.float32)
    acc_ref[...] += part

    @pl.when(g == HEAD_G - 1)
    def _():
        def leaky(v):
            return jnp.where(v > 0, v, ALPHA * v)
        d = leaky(acc_ref[...] + bd1_ref[...])
        d = leaky(jax.lax.dot_general(d, wd2_ref[...], (((1,), (0,)), ((), ())),
                                      preferred_element_type=jnp.float32) + bd2_ref[...])
        d = leaky(jax.lax.dot_general(d, wd3_ref[...], (((1,), (0,)), ((), ())),
                                      preferred_element_type=jnp.float32) + bd3_ref[...])
        o = jax.lax.dot_general(d, wd4_ref[...], (((1,), (0,)), ((), ())),
                                preferred_element_type=jnp.float32) + bd4_ref[...]
        out_ref[...] = o


def _mlp_head(flat, Wd1, bd1, Wd2, bd2, Wd3, bd3, Wd4, bd4):
    flat2 = flat.reshape(HEAD_G, 1, HEAD_BK)
    out = pl.pallas_call(
        _head_body,
        grid=(HEAD_G,),
        in_specs=[
            pl.BlockSpec((1, 1, HEAD_BK), lambda g: (g, 0, 0)),
            pl.BlockSpec((HEAD_BK, EMB), lambda g: (g, 0)),
            pl.BlockSpec((1, HID), lambda g: (0, 0)),
            pl.BlockSpec((HID, HID), lambda g: (0, 0)),
            pl.BlockSpec((1, HID), lambda g: (0, 0)),
            pl.BlockSpec((HID, HID), lambda g: (0, 0)),
            pl.BlockSpec((1, HID), lambda g: (0, 0)),
            pl.BlockSpec((HID, 1), lambda g: (0, 0)),
            pl.BlockSpec((1, 1), lambda g: (0, 0)),
        ],
        out_specs=pl.BlockSpec((1, 1), lambda g: (0, 0)),
        out_shape=jax.ShapeDtypeStruct((1, 1), jnp.float32),
        scratch_shapes=[pltpu.VMEM((1, HID), jnp.float32)],
    )(flat2, Wd1, bd1.reshape(1, HID), Wd2, bd2.reshape(1, HID),
      Wd3, bd3.reshape(1, HID), Wd4, bd4.reshape(1, 1))
    return out.reshape(1)


def kernel(x, edge_index, W1, b1, W2, b2, W3, b3, Wd1, bd1, Wd2, bd2, Wd3, bd3, Wd4, bd4):
    n = x.shape[0]
    src = edge_index[0]
    dst = edge_index[1]
    deg = jax.ops.segment_sum(jnp.ones_like(src, dtype=x.dtype), dst, num_segments=n) + 1.0
    dinv = jax.lax.rsqrt(deg)

    def conv(h, W, b):
        hs = (h @ W) * dinv[:, None]
        seg = jax.ops.segment_sum(hs[src], dst, num_segments=n)
        return jnp.tanh(dinv[:, None] * (seg + hs) + b)

    h = conv(x, W1, b1)
    h = conv(h, W2, b2)
    h = conv(h, W3, b3)
    return _mlp_head(h.reshape(-1), Wd1, bd1, Wd2, bd2, Wd3, bd3, Wd4, bd4)


# speedup vs baseline: 11.4263x; 4.3227x over previous
import functools

import jax
import jax.numpy as jnp
from jax import lax
from jax.experimental import pallas as pl
from jax.experimental.pallas import tpu as pltpu
from jax.experimental.pallas import tpu_sc as plsc

N = 10000
E = 320000
D_IN = 128
EMB = 64
HID = 64
ALPHA = 0.1

NC, NS = 2, 16
NW = NC * NS
CHUNK = 128
GCH = 80
EPAD = NW * GCH * CHUNK
NBUF = 8
LAG = 4
NPAD = 10112
RPT = NPAD // NS
DEGW = 16

HEAD_BK = 25600
HEAD_G = (N * EMB) // HEAD_BK

_mesh = plsc.VectorSubcoreMesh(core_axis_name="c", subcore_axis_name="s")
_sc_params = pltpu.CompilerParams(use_tc_tiling_on_sc=False)


def _deg_body(dstp_hbm, onehot_hbm, zeros_hbm, out_hbm, dst_buf, ones_buf,
              acc, sem_s):
    c = lax.axis_index("c")
    s = lax.axis_index("s")
    wid = s * NC + c
    row0 = s * RPT

    pltpu.sync_copy(zeros_hbm, acc.at[pl.ds(row0, RPT)])
    pltpu.sync_copy(dstp_hbm.at[wid], dst_buf)
    pltpu.sync_copy(onehot_hbm, ones_buf)
    plsc.subcore_barrier()

    def scat(g, b):
        pltpu.async_copy(ones_buf, acc.at[dst_buf.at[g]], sem_s.at[b],
                         add=True)

    def wait_s(b):
        pltpu.make_async_copy(ones_buf, acc.at[dst_buf.at[0]],
                              sem_s.at[b]).wait()

    for b in range(NBUF):
        scat(b, b)

    def ring(o, carry):
        for b in range(NBUF):
            wait_s(b)
            scat(o * NBUF + b, b)
        return carry

    lax.fori_loop(1, GCH // NBUF, ring, 0)
    for b in range(NBUF):
        wait_s(b)
    plsc.subcore_barrier()
    pltpu.sync_copy(acc.at[pl.ds(row0, RPT)], out_hbm.at[c, pl.ds(row0, RPT)])


_deg_call = pl.kernel(
    _deg_body,
    out_type=jax.ShapeDtypeStruct((NC, NPAD, DEGW), jnp.float32),
    mesh=_mesh,
    scratch_types=[
        pltpu.VMEM((GCH, CHUNK), jnp.int32),
        pltpu.VMEM((CHUNK, DEGW), jnp.float32),
        pltpu.VMEM_SHARED((NPAD, DEGW), jnp.float32),
        pltpu.SemaphoreType.DMA((NBUF,)),
    ],
    compiler_params=_sc_params,
)


def _conv_body(hs_hbm, srcp_hbm, dstp_hbm, zeros_hbm, out_hbm,
               src_buf, dst_buf, rows, acc, sem_g, sem_s):
    c = lax.axis_index("c")
    s = lax.axis_index("s")
    wid = s * NC + c
    row0 = s * RPT

    pltpu.sync_copy(zeros_hbm, acc.at[pl.ds(row0, RPT)])
    pltpu.sync_copy(srcp_hbm.at[wid], src_buf)
    pltpu.sync_copy(dstp_hbm.at[wid], dst_buf)
    plsc.subcore_barrier()

    def gath(g, b):
        pltpu.async_copy(hs_hbm.at[src_buf.at[g]], rows.at[b], sem_g.at[b])

    def scat(g, b):
        pltpu.async_copy(rows.at[b], acc.at[dst_buf.at[g]], sem_s.at[b],
                         add=True)

    def wait_g(b):
        pltpu.make_async_copy(hs_hbm.at[src_buf.at[0]], rows.at[b],
                              sem_g.at[b]).wait()

    def wait_s(b):
        pltpu.make_async_copy(rows.at[b], acc.at[dst_buf.at[0]],
                              sem_s.at[b]).wait()

    for i in range(LAG):
        gath(i, i)
    for i in range(LAG, NBUF):
        gath(i, i)
        wait_g(i - LAG)
        scat(i - LAG, i - LAG)

    def ring(o, carry):
        for b in range(NBUF):
            i = o * NBUF + b
            wait_s(b)
            gath(i, b)
            bj = (b + NBUF - LAG) % NBUF
            wait_g(bj)
            scat(i - LAG, bj)
        return carry

    lax.fori_loop(1, GCH // NBUF, ring, 0)

    for j in range(GCH - LAG, GCH):
        b = j % NBUF
        wait_g(b)
        scat(j, b)
    for b in range(NBUF):
        wait_s(b)
    plsc.subcore_barrier()
    pltpu.sync_copy(acc.at[pl.ds(row0, RPT)], out_hbm.at[c, pl.ds(row0, RPT)])


_conv_call = pl.kernel(
    _conv_body,
    out_type=jax.ShapeDtypeStruct((NC, NPAD, EMB), jnp.float32),
    mesh=_mesh,
    scratch_types=[
        pltpu.VMEM((GCH, CHUNK), jnp.int32),
        pltpu.VMEM((GCH, CHUNK), jnp.int32),
        pltpu.VMEM((NBUF, CHUNK, EMB), jnp.float32),
        pltpu.VMEM_SHARED((NPAD, EMB), jnp.float32),
        pltpu.SemaphoreType.DMA((NBUF,)),
        pltpu.SemaphoreType.DMA((NBUF,)),
    ],
    compiler_params=_sc_params,
)


def _tca_body(degp_ref, x_ref, w_ref, dinv_ref, hs_ref):
    d = degp_ref[0, :, 0:1] + degp_ref[1, :, 0:1] + 1.0
    dinv = lax.rsqrt(d)
    dinv_ref[...] = dinv
    h = jax.lax.dot_general(x_ref[...], w_ref[...], (((1,), (0,)), ((), ())),
                            preferred_element_type=jnp.float32)
    hs_ref[...] = h * dinv[:N]


def _tca(degp, x, W1):
    return pl.pallas_call(
        _tca_body,
        out_shape=(jax.ShapeDtypeStruct((NPAD, 1), jnp.float32),
                   jax.ShapeDtypeStruct((N, EMB), jnp.float32)),
    )(degp, x, W1)


def _tcb_body(p_ref, hs_ref, dinv_ref, b_ref, w_ref, out_ref):
    dv = dinv_ref[...][:N]
    seg = p_ref[0, :N] + p_ref[1, :N]
    t = jnp.tanh((seg + hs_ref[...]) * dv + b_ref[...])
    out_ref[...] = jax.lax.dot_general(
        t, w_ref[...], (((1,), (0,)), ((), ())),
        preferred_element_type=jnp.float32) * dv


def _tcb(P, hs, dinv_col, b, Wn):
    return pl.pallas_call(
        _tcb_body,
        out_shape=jax.ShapeDtypeStruct((N, EMB), jnp.float32),
    )(P, hs, dinv_col, b.reshape(1, EMB), Wn)


def _tcc_body(p_ref, hs_ref, dinv_ref, b_ref, out_ref):
    dv = dinv_ref[...][:N]
    seg = p_ref[0, :N] + p_ref[1, :N]
    out_ref[...] = jnp.tanh((seg + hs_ref[...]) * dv + b_ref[...])


def _tcc(P, hs, dinv_col, b):
    return pl.pallas_call(
        _tcc_body,
        out_shape=jax.ShapeDtypeStruct((N, EMB), jnp.float32),
    )(P, hs, dinv_col, b.reshape(1, EMB))


def _head_body(flat_ref, wd1_ref, bd1_ref, wd2_ref, bd2_ref, wd3_ref, bd3_ref,
               wd4_ref, bd4_ref, out_ref, acc_ref):
    g = pl.program_id(0)

    @pl.when(g == 0)
    def _():
        acc_ref[...] = jnp.zeros_like(acc_ref)

    part = jax.lax.dot_general(
        flat_ref[0], wd1_ref[...],
        dimension_numbers=(((1,), (0,)), ((), ())),
        preferred_element_type=jnp.float32)
    acc_ref[...] += part

    @pl.when(g == HEAD_G - 1)
    def _():
        def leaky(v):
            return jnp.where(v > 0, v, ALPHA * v)
        d = leaky(acc_ref[...] + bd1_ref[...])
        d = leaky(jax.lax.dot_general(d, wd2_ref[...], (((1,), (0,)), ((), ())),
                                      preferred_element_type=jnp.float32) + bd2_ref[...])
        d = leaky(jax.lax.dot_general(d, wd3_ref[...], (((1,), (0,)), ((), ())),
                                      preferred_element_type=jnp.float32) + bd3_ref[...])
        o = jax.lax.dot_general(d, wd4_ref[...], (((1,), (0,)), ((), ())),
                                preferred_element_type=jnp.float32) + bd4_ref[...]
        out_ref[...] = o


def _mlp_head(flat, Wd1, bd1, Wd2, bd2, Wd3, bd3, Wd4, bd4):
    flat2 = flat.reshape(HEAD_G, 1, HEAD_BK)
    out = pl.pallas_call(
        _head_body,
        grid=(HEAD_G,),
        in_specs=[
            pl.BlockSpec((1, 1, HEAD_BK), lambda g: (g, 0, 0)),
            pl.BlockSpec((HEAD_BK, EMB), lambda g: (g, 0)),
            pl.BlockSpec((1, HID), lambda g: (0, 0)),
            pl.BlockSpec((HID, HID), lambda g: (0, 0)),
            pl.BlockSpec((1, HID), lambda g: (0, 0)),
            pl.BlockSpec((HID, HID), lambda g: (0, 0)),
            pl.BlockSpec((1, HID), lambda g: (0, 0)),
            pl.BlockSpec((HID, 1), lambda g: (0, 0)),
            pl.BlockSpec((1, 1), lambda g: (0, 0)),
        ],
        out_specs=pl.BlockSpec((1, 1), lambda g: (0, 0)),
        out_shape=jax.ShapeDtypeStruct((1, 1), jnp.float32),
        scratch_shapes=[pltpu.VMEM((1, HID), jnp.float32)],
    )(flat2, Wd1, bd1.reshape(1, HID), Wd2, bd2.reshape(1, HID),
      Wd3, bd3.reshape(1, HID), Wd4, bd4.reshape(1, 1))
    return out.reshape(1)


def kernel(x, edge_index, W1, b1, W2, b2, W3, b3, Wd1, bd1, Wd2, bd2, Wd3, bd3, Wd4, bd4):
    src = edge_index[0]
    dst = edge_index[1]
    pad = EPAD - E
    srcp = jnp.concatenate([src, jnp.zeros((pad,), src.dtype)]).reshape(NW, GCH, CHUNK)
    dstp = jnp.concatenate([dst, jnp.full((pad,), N, dst.dtype)]).reshape(NW, GCH, CHUNK)
    zeros64 = jnp.zeros((RPT, EMB), jnp.float32)
    zeros16 = jnp.zeros((RPT, DEGW), jnp.float32)
    onehot = jnp.zeros((CHUNK, DEGW), jnp.float32).at[:, 0].set(1.0)

    degp = _deg_call(dstp, onehot, zeros16)
    dinv_col, hs = _tca(degp, x, W1)

    P = _conv_call(hs, srcp, dstp, zeros64)
    hs = _tcb(P, hs, dinv_col, b1, W2)
    P = _conv_call(hs, srcp, dstp, zeros64)
    hs = _tcb(P, hs, dinv_col, b2, W3)
    P = _conv_call(hs, srcp, dstp, zeros64)
    h3 = _tcc(P, hs, dinv_col, b3)

    return _mlp_head(h3.reshape(-1), Wd1, bd1, Wd2, bd2, Wd3, bd3, Wd4, bd4)


# speedup vs baseline: 17.5880x; 1.5392x over previous
import functools

import jax
import jax.numpy as jnp
from jax import lax
from jax.experimental import pallas as pl
from jax.experimental.pallas import tpu as pltpu
from jax.experimental.pallas import tpu_sc as plsc

N = 10000
E = 320000
D_IN = 128
EMB = 64
HID = 64
ALPHA = 0.1

NC, NS = 2, 16
NW = NC * NS
CHUNK = 128
GCH = 80
EPAD = NW * GCH * CHUNK
NBUF = 8
LAG = 4
NPAD = 10112
RPT = NPAD // NS
DEGW = 16

HEAD_BK = 25600
HEAD_G = (N * EMB) // HEAD_BK

_mesh = plsc.VectorSubcoreMesh(core_axis_name="c", subcore_axis_name="s")
_sc_params = pltpu.CompilerParams(use_tc_tiling_on_sc=False)


def _deg_body(dstp_hbm, onehot_hbm, zeros_hbm, out_hbm, dst_buf, ones_buf,
              acc, sem_s):
    c = lax.axis_index("c")
    s = lax.axis_index("s")
    wid = s * NC + c
    row0 = s * RPT

    pltpu.sync_copy(zeros_hbm, acc.at[pl.ds(row0, RPT)])
    pltpu.sync_copy(dstp_hbm.at[wid], dst_buf)
    pltpu.sync_copy(onehot_hbm, ones_buf)
    plsc.subcore_barrier()

    def scat(g, b):
        pltpu.async_copy(ones_buf, acc.at[dst_buf.at[g]], sem_s.at[b],
                         add=True)

    def wait_s(b):
        pltpu.make_async_copy(ones_buf, acc.at[dst_buf.at[0]],
                              sem_s.at[b]).wait()

    for b in range(NBUF):
        scat(b, b)

    def ring(o, carry):
        for b in range(NBUF):
            wait_s(b)
            scat(o * NBUF + b, b)
        return carry

    lax.fori_loop(1, GCH // NBUF, ring, 0)
    for b in range(NBUF):
        wait_s(b)
    plsc.subcore_barrier()
    pltpu.sync_copy(acc.at[pl.ds(row0, RPT)], out_hbm.at[c, pl.ds(row0, RPT)])


_deg_call = pl.kernel(
    _deg_body,
    out_type=jax.ShapeDtypeStruct((NC, NPAD, DEGW), jnp.float32),
    mesh=_mesh,
    scratch_types=[
        pltpu.VMEM((GCH, CHUNK), jnp.int32),
        pltpu.VMEM((CHUNK, DEGW), jnp.float32),
        pltpu.VMEM_SHARED((NPAD, DEGW), jnp.float32),
        pltpu.SemaphoreType.DMA((NBUF,)),
    ],
    compiler_params=_sc_params,
)


HEMB = EMB // 2


def _conv_body(hs_hbm, srcp_hbm, dstp_hbm, zeros_hbm, out_hbm,
               src_buf, dst_buf, rows, acc, hs_sp, sem_g, sem_s):
    c = lax.axis_index("c")
    s = lax.axis_index("s")
    wid = s * NC + c
    row0 = s * RPT

    pltpu.sync_copy(srcp_hbm.at[wid], src_buf)
    pltpu.sync_copy(dstp_hbm.at[wid], dst_buf)

    def gath(g, b):
        pltpu.async_copy(hs_sp.at[src_buf.at[g]], rows.at[b], sem_g.at[b])

    def scat(g, b):
        pltpu.async_copy(rows.at[b], acc.at[dst_buf.at[g]], sem_s.at[b],
                         add=True)

    def wait_g(b):
        pltpu.make_async_copy(hs_sp.at[src_buf.at[0]], rows.at[b],
                              sem_g.at[b]).wait()

    def wait_s(b):
        pltpu.make_async_copy(rows.at[b], acc.at[dst_buf.at[0]],
                              sem_s.at[b]).wait()

    for h in range(2):
        pltpu.sync_copy(zeros_hbm, acc.at[pl.ds(row0, RPT)])
        pltpu.sync_copy(hs_hbm.at[h, pl.ds(row0, RPT)],
                        hs_sp.at[pl.ds(row0, RPT)])
        plsc.subcore_barrier()

        for i in range(LAG):
            gath(i, i)
        for i in range(LAG, NBUF):
            gath(i, i)
            wait_g(i - LAG)
            scat(i - LAG, i - LAG)

        def ring(o, carry):
            for b in range(NBUF):
                i = o * NBUF + b
                wait_s(b)
                gath(i, b)
                bj = (b + NBUF - LAG) % NBUF
                wait_g(bj)
                scat(i - LAG, bj)
            return carry

        lax.fori_loop(1, GCH // NBUF, ring, 0)

        for j in range(GCH - LAG, GCH):
            b = j % NBUF
            wait_g(b)
            scat(j, b)
        for b in range(NBUF):
            wait_s(b)
        plsc.subcore_barrier()
        pltpu.sync_copy(acc.at[pl.ds(row0, RPT)],
                        out_hbm.at[c, h, pl.ds(row0, RPT)])


_conv_call = pl.kernel(
    _conv_body,
    out_type=jax.ShapeDtypeStruct((NC, 2, NPAD, HEMB), jnp.float32),
    mesh=_mesh,
    scratch_types=[
        pltpu.VMEM((GCH, CHUNK), jnp.int32),
        pltpu.VMEM((GCH, CHUNK), jnp.int32),
        pltpu.VMEM((NBUF, CHUNK, HEMB), jnp.float32),
        pltpu.VMEM_SHARED((NPAD, HEMB), jnp.float32),
        pltpu.VMEM_SHARED((NPAD, HEMB), jnp.float32),
        pltpu.SemaphoreType.DMA((NBUF,)),
        pltpu.SemaphoreType.DMA((NBUF,)),
    ],
    compiler_params=_sc_params,
)


def _tca_body(degp_ref, x_ref, w_ref, dinv_ref, hs_ref):
    d = degp_ref[0, :, 0:1] + degp_ref[1, :, 0:1] + 1.0
    dinv = lax.rsqrt(d)
    dinv_ref[...] = dinv
    h = jax.lax.dot_general(x_ref[...], w_ref[...], (((1,), (0,)), ((), ())),
                            preferred_element_type=jnp.float32)
    hs = h * dinv[:N]
    hs_ref[0, 0:N] = hs[:, :HEMB]
    hs_ref[1, 0:N] = hs[:, HEMB:]
    hs_ref[0, N:NPAD] = jnp.zeros((NPAD - N, HEMB), jnp.float32)
    hs_ref[1, N:NPAD] = jnp.zeros((NPAD - N, HEMB), jnp.float32)


def _tca(degp, x, W1):
    return pl.pallas_call(
        _tca_body,
        out_shape=(jax.ShapeDtypeStruct((NPAD, 1), jnp.float32),
                   jax.ShapeDtypeStruct((2, NPAD, HEMB), jnp.float32)),
    )(degp, x, W1)


def _seg_hs(p_ref, hs_ref):
    seg0 = p_ref[0, 0, :N] + p_ref[1, 0, :N] + hs_ref[0, :N]
    seg1 = p_ref[0, 1, :N] + p_ref[1, 1, :N] + hs_ref[1, :N]
    return jnp.concatenate([seg0, seg1], axis=1)


def _tcb_body(p_ref, hs_ref, dinv_ref, b_ref, w_ref, out_ref):
    dv = dinv_ref[...][:N]
    t = jnp.tanh(_seg_hs(p_ref, hs_ref) * dv + b_ref[...])
    nxt = jax.lax.dot_general(
        t, w_ref[...], (((1,), (0,)), ((), ())),
        preferred_element_type=jnp.float32) * dv
    out_ref[0, 0:N] = nxt[:, :HEMB]
    out_ref[1, 0:N] = nxt[:, HEMB:]
    out_ref[0, N:NPAD] = jnp.zeros((NPAD - N, HEMB), jnp.float32)
    out_ref[1, N:NPAD] = jnp.zeros((NPAD - N, HEMB), jnp.float32)


def _tcb(P, hs, dinv_col, b, Wn):
    return pl.pallas_call(
        _tcb_body,
        out_shape=jax.ShapeDtypeStruct((2, NPAD, HEMB), jnp.float32),
    )(P, hs, dinv_col, b.reshape(1, EMB), Wn)


def _tcc_body(p_ref, hs_ref, dinv_ref, b_ref, out_ref):
    dv = dinv_ref[...][:N]
    out_ref[...] = jnp.tanh(_seg_hs(p_ref, hs_ref) * dv + b_ref[...])


def _tcc(P, hs, dinv_col, b):
    return pl.pallas_call(
        _tcc_body,
        out_shape=jax.ShapeDtypeStruct((N, EMB), jnp.float32),
    )(P, hs, dinv_col, b.reshape(1, EMB))


def _head_body(flat_ref, wd1_ref, bd1_ref, wd2_ref, bd2_ref, wd3_ref, bd3_ref,
               wd4_ref, bd4_ref, out_ref, acc_ref):
    g = pl.program_id(0)

    @pl.when(g == 0)
    def _():
        acc_ref[...] = jnp.zeros_like(acc_ref)

    part = jax.lax.dot_general(
        flat_ref[0], wd1_ref[...],
        dimension_numbers=(((1,), (0,)), ((), ())),
        preferred_element_type=jnp.float32)
    acc_ref[...] += part

    @pl.when(g == HEAD_G - 1)
    def _():
        def leaky(v):
            return jnp.where(v > 0, v, ALPHA * v)
        d = leaky(acc_ref[...] + bd1_ref[...])
        d = leaky(jax.lax.dot_general(d, wd2_ref[...], (((1,), (0,)), ((), ())),
                                      preferred_element_type=jnp.float32) + bd2_ref[...])
        d = leaky(jax.lax.dot_general(d, wd3_ref[...], (((1,), (0,)), ((), ())),
                                      preferred_element_type=jnp.float32) + bd3_ref[...])
        o = jax.lax.dot_general(d, wd4_ref[...], (((1,), (0,)), ((), ())),
                                preferred_element_type=jnp.float32) + bd4_ref[...]
        out_ref[...] = o


def _mlp_head(flat, Wd1, bd1, Wd2, bd2, Wd3, bd3, Wd4, bd4):
    flat2 = flat.reshape(HEAD_G, 1, HEAD_BK)
    out = pl.pallas_call(
        _head_body,
        grid=(HEAD_G,),
        in_specs=[
            pl.BlockSpec((1, 1, HEAD_BK), lambda g: (g, 0, 0)),
            pl.BlockSpec((HEAD_BK, EMB), lambda g: (g, 0)),
            pl.BlockSpec((1, HID), lambda g: (0, 0)),
            pl.BlockSpec((HID, HID), lambda g: (0, 0)),
            pl.BlockSpec((1, HID), lambda g: (0, 0)),
            pl.BlockSpec((HID, HID), lambda g: (0, 0)),
            pl.BlockSpec((1, HID), lambda g: (0, 0)),
            pl.BlockSpec((HID, 1), lambda g: (0, 0)),
            pl.BlockSpec((1, 1), lambda g: (0, 0)),
        ],
        out_specs=pl.BlockSpec((1, 1), lambda g: (0, 0)),
        out_shape=jax.ShapeDtypeStruct((1, 1), jnp.float32),
        scratch_shapes=[pltpu.VMEM((1, HID), jnp.float32)],
    )(flat2, Wd1, bd1.reshape(1, HID), Wd2, bd2.reshape(1, HID),
      Wd3, bd3.reshape(1, HID), Wd4, bd4.reshape(1, 1))
    return out.reshape(1)


def kernel(x, edge_index, W1, b1, W2, b2, W3, b3, Wd1, bd1, Wd2, bd2, Wd3, bd3, Wd4, bd4):
    src = edge_index[0]
    dst = edge_index[1]
    pad = EPAD - E
    srcp = jnp.concatenate([src, jnp.zeros((pad,), src.dtype)]).reshape(NW, GCH, CHUNK)
    dstp = jnp.concatenate([dst, jnp.full((pad,), N, dst.dtype)]).reshape(NW, GCH, CHUNK)
    zeros64 = jnp.zeros((RPT, HEMB), jnp.float32)
    zeros16 = jnp.zeros((RPT, DEGW), jnp.float32)
    onehot = jnp.zeros((CHUNK, DEGW), jnp.float32).at[:, 0].set(1.0)

    degp = _deg_call(dstp, onehot, zeros16)
    dinv_col, hs = _tca(degp, x, W1)

    P = _conv_call(hs, srcp, dstp, zeros64)
    hs = _tcb(P, hs, dinv_col, b1, W2)
    P = _conv_call(hs, srcp, dstp, zeros64)
    hs = _tcb(P, hs, dinv_col, b2, W3)
    P = _conv_call(hs, srcp, dstp, zeros64)
    h3 = _tcc(P, hs, dinv_col, b3)

    return _mlp_head(h3.reshape(-1), Wd1, bd1, Wd2, bd2, Wd3, bd3, Wd4, bd4)
